# baseline (speedup 1.0000x reference)
import functools

import jax
import jax.numpy as jnp
from jax import lax
from jax.experimental import pallas as pl
from jax.experimental.pallas import tpu as pltpu
from jax.experimental.pallas import tpu_sc as plsc

_NE = 1024
_D = 64
_BETA = 0.25

_SC_CORES = 2
_SC_SUBCORES = 16
_SC_WORKERS = _SC_CORES * _SC_SUBCORES


def _dist_argmin_body(z_ref, emb_ref, idx_ref, loss_ref, acc_ref):
    i = pl.program_id(0)
    z = z_ref[0]
    emb = emb_ref[...]
    scores = lax.dot_general(
        z, emb, (((1,), (1,)), ((), ())), preferred_element_type=jnp.float32
    )
    z_sq = jnp.sum(z**2, axis=1, keepdims=True)
    e_sq = jnp.sum(emb**2, axis=1)
    dist = z_sq - 2.0 * scores + e_sq[None, :]
    dmin = jnp.min(dist, axis=1, keepdims=True)
    eids = lax.broadcasted_iota(jnp.int32, dist.shape, 1)
    idx = jnp.min(jnp.where(dist == dmin, eids, _NE), axis=1)
    idx_ref[0, 0, :] = idx
    part = jnp.sum(dmin[:, 0])

    @pl.when(i == 0)
    def _init():
        acc_ref[0] = part

    @pl.when(i > 0)
    def _accum():
        acc_ref[0] += part

    @pl.when(i == pl.num_programs(0) - 1)
    def _fin():
        n_elems = pl.num_programs(0) * z.shape[0] * z.shape[1]
        loss_ref[0, 0] = acc_ref[0] * ((1.0 + _BETA) / n_elems)


def _dist_argmin(z, emb_weight):
    B, T, D = z.shape
    return pl.pallas_call(
        _dist_argmin_body,
        grid=(B,),
        in_specs=[
            pl.BlockSpec((1, T, D), lambda i: (i, 0, 0)),
            pl.BlockSpec((_NE, D), lambda i: (0, 0)),
        ],
        out_specs=[
            pl.BlockSpec((1, 1, T), lambda i: (i, 0, 0)),
            pl.BlockSpec(memory_space=pltpu.SMEM),
        ],
        out_shape=[
            jax.ShapeDtypeStruct((B, 1, T), jnp.int32),
            jax.ShapeDtypeStruct((1, 1), jnp.float32),
        ],
        scratch_shapes=[pltpu.SMEM((1,), jnp.float32)],
    )(z, emb_weight)


def _make_sc_gather(n_rows):
    bpw = n_rows // _SC_WORKERS
    mesh = plsc.VectorSubcoreMesh(core_axis_name="c", subcore_axis_name="s")

    @functools.partial(
        pl.kernel,
        mesh=mesh,
        out_type=jax.ShapeDtypeStruct((n_rows, _D), jnp.float32),
        scratch_types=[
            pltpu.VMEM((bpw,), jnp.int32),
            pltpu.VMEM((bpw, _D), jnp.float32),
            pltpu.SemaphoreType.DMA,
        ],
        compiler_params=pltpu.CompilerParams(use_tc_tiling_on_sc=False),
    )
    def gather(emb_hbm, idx_hbm, out_hbm, idx_v, rows_v, sem):
        wid = lax.axis_index("s") * _SC_CORES + lax.axis_index("c")
        base = wid * bpw
        pltpu.sync_copy(idx_hbm.at[pl.ds(base, bpw)], idx_v)
        pltpu.async_copy(emb_hbm.at[idx_v], rows_v, sem).wait()
        pltpu.sync_copy(rows_v, out_hbm.at[pl.ds(base, bpw)])

    return gather


def kernel(z, emb_weight):
    B, T, D = z.shape
    idx3, loss2 = _dist_argmin(z, emb_weight)
    idx_flat = idx3.reshape(B * T)
    z_q = _make_sc_gather(B * T)(emb_weight, idx_flat)
    return z_q.reshape(z.shape), loss2[0, 0], idx3.reshape(B, T)


# speedup vs baseline: 1.5736x; 1.5736x over previous
import jax
import jax.numpy as jnp
from jax import lax
from jax.experimental import pallas as pl
from jax.experimental.pallas import tpu as pltpu

_NE = 1024
_BETA = 0.25


def _vq_body(z_ref, emb_ref, zq_ref, idx_ref, loss_ref, acc_ref):
    i = pl.program_id(0)
    z = z_ref[0]
    emb = emb_ref[...]
    scores = lax.dot_general(
        z, emb, (((1,), (1,)), ((), ())), preferred_element_type=jnp.float32
    )
    z_sq = jnp.sum(z**2, axis=1, keepdims=True)
    e_sq = jnp.sum(emb**2, axis=1)
    dist = z_sq - 2.0 * scores + e_sq[None, :]
    dmin = jnp.min(dist, axis=1, keepdims=True)
    eids = lax.broadcasted_iota(jnp.int32, dist.shape, 1)
    idx = jnp.min(jnp.where(dist == dmin, eids, _NE), axis=1)
    idx_ref[0, 0, :] = idx
    onehot = jnp.where(eids == idx[:, None], 1.0, 0.0)
    z_q = lax.dot_general(
        onehot, emb, (((1,), (0,)), ((), ())),
        preferred_element_type=jnp.float32,
    )
    zq_ref[0] = z_q
    diff = z_q - z
    part = jnp.sum(diff * diff)

    @pl.when(i == 0)
    def _init():
        acc_ref[0] = part

    @pl.when(i > 0)
    def _accum():
        acc_ref[0] += part

    @pl.when(i == pl.num_programs(0) - 1)
    def _fin():
        n_elems = pl.num_programs(0) * z.shape[0] * z.shape[1]
        loss_ref[0, 0] = acc_ref[0] * ((1.0 + _BETA) / n_elems)


def kernel(z, emb_weight):
    B, T, D = z.shape
    z_q, idx3, loss2 = pl.pallas_call(
        _vq_body,
        grid=(B,),
        in_specs=[
            pl.BlockSpec((1, T, D), lambda i: (i, 0, 0)),
            pl.BlockSpec((_NE, D), lambda i: (0, 0)),
        ],
        out_specs=[
            pl.BlockSpec((1, T, D), lambda i: (i, 0, 0)),
            pl.BlockSpec((1, 1, T), lambda i: (i, 0, 0)),
            pl.BlockSpec(memory_space=pltpu.SMEM),
        ],
        out_shape=[
            jax.ShapeDtypeStruct((B, T, D), jnp.float32),
            jax.ShapeDtypeStruct((B, 1, T), jnp.int32),
            jax.ShapeDtypeStruct((1, 1), jnp.float32),
        ],
        scratch_shapes=[pltpu.SMEM((1,), jnp.float32)],
    )(z, emb_weight)
    return z_q, loss2[0, 0], idx3.reshape(B, T)


# speedup vs baseline: 1.9405x; 1.2332x over previous
import jax
import jax.numpy as jnp
from jax import lax
from jax.experimental import pallas as pl
from jax.experimental.pallas import tpu as pltpu

_NE = 1024
_BETA = 0.25


def _vq_body(z_ref, emb_ref, zq_ref, idx_ref, loss_ref, acc_ref):
    i = pl.program_id(0)
    z = z_ref[0]
    emb = emb_ref[...]
    s2 = lax.dot_general(
        z * -2.0, emb, (((1,), (1,)), ((), ())),
        preferred_element_type=jnp.float32,
    )
    z_sq = jnp.sum(z**2, axis=1, keepdims=True)
    e_sq = jnp.sum(emb**2, axis=1)
    dist = (z_sq + s2) + e_sq[None, :]
    dmin = jnp.min(dist, axis=1, keepdims=True)
    eids = lax.broadcasted_iota(jnp.int32, dist.shape, 1).astype(jnp.float32)
    idxf = jnp.min(jnp.where(dist == dmin, eids, float(_NE)), axis=1)
    onehot = jnp.where(eids == idxf[:, None], 1.0, 0.0)
    z_q = lax.dot_general(
        onehot, emb, (((1,), (0,)), ((), ())),
        preferred_element_type=jnp.float32,
    )
    zq_ref[0] = z_q
    iota_row = lax.broadcasted_iota(jnp.int32, (1, _NE), 1).astype(jnp.float32)
    idx_row = lax.dot_general(
        iota_row, onehot, (((1,), (1,)), ((), ())),
        preferred_element_type=jnp.float32,
    )
    idx_ref[pl.ds(i, 1), :] = idx_row.astype(jnp.int32)
    part = jnp.sum(dmin)

    @pl.when(i == 0)
    def _init():
        acc_ref[0] = part

    @pl.when(i > 0)
    def _accum():
        acc_ref[0] += part

    @pl.when(i == pl.num_programs(0) - 1)
    def _fin():
        n_elems = pl.num_programs(0) * z.shape[0] * z.shape[1]
        loss_ref[0, 0] = acc_ref[0] * ((1.0 + _BETA) / n_elems)


def kernel(z, emb_weight):
    B, T, D = z.shape
    z_q, idx2, loss2 = pl.pallas_call(
        _vq_body,
        grid=(B,),
        in_specs=[
            pl.BlockSpec((1, T, D), lambda i: (i, 0, 0)),
            pl.BlockSpec((_NE, D), lambda i: (0, 0)),
        ],
        out_specs=[
            pl.BlockSpec((1, T, D), lambda i: (i, 0, 0)),
            pl.BlockSpec((B, T), lambda i: (0, 0)),
            pl.BlockSpec(memory_space=pltpu.SMEM),
        ],
        out_shape=[
            jax.ShapeDtypeStruct((B, T, D), jnp.float32),
            jax.ShapeDtypeStruct((B, T), jnp.int32),
            jax.ShapeDtypeStruct((1, 1), jnp.float32),
        ],
        scratch_shapes=[pltpu.SMEM((1,), jnp.float32)],
    )(z, emb_weight)
    return z_q, loss2[0, 0], idx2
